# Initial kernel scaffold; baseline (speedup 1.0000x reference)
import jax
import jax.numpy as jnp
from jax.experimental import pallas as pl


def kernel(inputs, term_table, pos_table):
    raise NotImplementedError("write your pallas kernel here")


# speedup vs baseline: 4.4847x; 4.4847x over previous
import functools

import jax
import jax.numpy as jnp
from jax import lax
from jax.experimental import pallas as pl
from jax.experimental.pallas import tpu as pltpu
from jax.experimental.pallas import tpu_sc as plsc

SEQ = 200
DIM = 32
LANES = 16
HALF = DIM // LANES
K_ROWS = 4
F = SEQ * K_ROWS


@functools.lru_cache(maxsize=None)
def _build_sc_kernel(n_flat):
    info = plsc.get_sparse_core_info()
    nc, ns = info.num_cores, info.num_subcores
    nw = nc * ns
    per_w = n_flat // nw
    n_chunks = per_w // F
    assert per_w % F == 0 and n_flat % nw == 0

    mesh = plsc.VectorSubcoreMesh(core_axis_name="c", subcore_axis_name="s")

    @functools.partial(
        pl.kernel,
        mesh=mesh,
        compiler_params=pltpu.CompilerParams(use_tc_tiling_on_sc=False),
        out_type=jax.ShapeDtypeStruct((n_flat, DIM), jnp.float32),
        scratch_types=[
            pltpu.VMEM((F,), jnp.int32),
            pltpu.VMEM((F, DIM), jnp.float32),
            pltpu.VMEM((SEQ, DIM), jnp.float32),
            pltpu.SemaphoreType.DMA,
        ],
    )
    def sc_kernel(idx_hbm, term_hbm, pos_hbm, out_hbm, idx_v, rows_v, pos_v, sem):
        wid = lax.axis_index("s") * nc + lax.axis_index("c")
        base_w = wid * per_w
        pltpu.sync_copy(pos_hbm, pos_v)

        def chunk_body(c, _):
            base = base_w + c * F
            pltpu.sync_copy(idx_hbm.at[pl.ds(base, F)], idx_v)
            pltpu.async_copy(term_hbm.at[idx_v], rows_v, sem).wait()

            def add_row(r, _):
                for h in range(HALF):
                    p = pos_v[r, pl.ds(h * LANES, LANES)]
                    for j in range(K_ROWS):
                        rows_v[r + j * SEQ, pl.ds(h * LANES, LANES)] += p
                return 0

            lax.fori_loop(0, SEQ, add_row, 0)
            pltpu.sync_copy(rows_v, out_hbm.at[pl.ds(base, F)])
            return 0

        lax.fori_loop(0, n_chunks, chunk_body, 0)

    return sc_kernel


def kernel(inputs, term_table, pos_table):
    b, l = inputs.shape
    flat_idx = inputs.reshape(-1)
    out = _build_sc_kernel(flat_idx.shape[0])(flat_idx, term_table, pos_table)
    return out.reshape(b, l, DIM)


# speedup vs baseline: 5.0180x; 1.1189x over previous
import functools

import jax
import jax.numpy as jnp
from jax import lax
from jax.experimental import pallas as pl
from jax.experimental.pallas import tpu as pltpu
from jax.experimental.pallas import tpu_sc as plsc

SEQ = 200
DIM = 32
LANES = 16
HALF = DIM // LANES
K_ROWS = 4
F = SEQ * K_ROWS
NBUF = 4


@functools.lru_cache(maxsize=None)
def _build_sc_kernel(n_flat):
    info = plsc.get_sparse_core_info()
    nc, ns = info.num_cores, info.num_subcores
    nw = nc * ns
    per_w = n_flat // nw
    n_chunks = per_w // F
    assert per_w % F == 0 and n_flat % nw == 0 and n_chunks % NBUF == 0

    mesh = plsc.VectorSubcoreMesh(core_axis_name="c", subcore_axis_name="s")

    @functools.partial(
        pl.kernel,
        mesh=mesh,
        compiler_params=pltpu.CompilerParams(use_tc_tiling_on_sc=False),
        out_type=jax.ShapeDtypeStruct((n_flat, DIM), jnp.float32),
        scratch_types=[
            [pltpu.VMEM((F,), jnp.int32) for _ in range(NBUF)],
            [pltpu.VMEM((F, DIM), jnp.float32) for _ in range(NBUF)],
            pltpu.VMEM((SEQ, DIM), jnp.float32),
            [pltpu.SemaphoreType.DMA for _ in range(NBUF)],
            [pltpu.SemaphoreType.DMA for _ in range(NBUF)],
        ],
    )
    def sc_kernel(idx_hbm, term_hbm, pos_hbm, out_hbm,
                  idx_v, rows_v, pos_v, gsems, ssems):
        wid = lax.axis_index("s") * nc + lax.axis_index("c")
        base_w = wid * per_w
        pltpu.sync_copy(pos_hbm, pos_v)

        def issue_gather(c, b):
            base = base_w + c * F
            pltpu.sync_copy(idx_hbm.at[pl.ds(base, F)], idx_v[b])
            pltpu.async_copy(term_hbm.at[idx_v[b]], rows_v[b], gsems[b])

        def wait_gather(b):
            pltpu.make_async_copy(term_hbm.at[idx_v[b]], rows_v[b], gsems[b]).wait()

        def issue_store(c, b):
            base = base_w + c * F
            pltpu.async_copy(rows_v[b], out_hbm.at[pl.ds(base, F)], ssems[b])

        def wait_store(b):
            pltpu.make_async_copy(
                rows_v[b], out_hbm.at[pl.ds(base_w, F)], ssems[b]).wait()

        def add_pos(b):
            def add_row(r, _):
                for h in range(HALF):
                    p = pos_v[r, pl.ds(h * LANES, LANES)]
                    for j in range(K_ROWS):
                        rows_v[b][r + j * SEQ, pl.ds(h * LANES, LANES)] += p
                return 0
            lax.fori_loop(0, SEQ, add_row, 0)

        issue_gather(0, 0)

        def group_body(g, _):
            for b in range(NBUF):
                c = g * NBUF + b
                bn = (b + 1) % NBUF
                cn = c + 1

                @pl.when(cn < n_chunks)
                def _():
                    @pl.when(c >= NBUF - 1)
                    def _():
                        wait_store(bn)
                    issue_gather(cn, bn)

                wait_gather(b)
                add_pos(b)
                issue_store(c, b)
            return 0

        lax.fori_loop(0, n_chunks // NBUF, group_body, 0)

        for b in range(NBUF):
            wait_store(b)

    return sc_kernel


def kernel(inputs, term_table, pos_table):
    b, l = inputs.shape
    flat_idx = inputs.reshape(-1)
    out = _build_sc_kernel(flat_idx.shape[0])(flat_idx, term_table, pos_table)
    return out.reshape(b, l, DIM)
